# initial kernel scaffold (unmeasured)
import jax
import jax.numpy as jnp
from jax import lax
from jax.experimental import pallas as pl
from jax.experimental.pallas import tpu as pltpu


def kernel(
    x,
):
    def body(*refs):
        pass

    out_shape = jax.ShapeDtypeStruct(..., jnp.float32)
    return pl.pallas_call(body, out_shape=out_shape)(...)


# baseline (device time: 27978 ns/iter reference)
import jax
import jax.numpy as jnp
from jax import lax
from jax.experimental import pallas as pl
from jax.experimental.pallas import tpu as pltpu

N_DEV = 16
LOG2_N = 4


def kernel(x):
    m, n = x.shape

    def body(x_ref, out_ref, recv_ref, send_sems, recv_sems):
        my = lax.axis_index("i")

        barrier = pltpu.get_barrier_semaphore()
        for k in range(LOG2_N):
            partner = my ^ (1 << k)
            pl.semaphore_signal(
                barrier,
                inc=1,
                device_id=(partner,),
                device_id_type=pl.DeviceIdType.MESH,
            )
        pl.semaphore_wait(barrier, LOG2_N)

        out_ref[...] = x_ref[...]
        for k in range(LOG2_N):
            partner = my ^ (1 << k)
            rdma = pltpu.make_async_remote_copy(
                src_ref=out_ref,
                dst_ref=recv_ref.at[k],
                send_sem=send_sems.at[k],
                recv_sem=recv_sems.at[k],
                device_id=(partner,),
                device_id_type=pl.DeviceIdType.MESH,
            )
            rdma.start()
            rdma.wait()
            out_ref[...] = out_ref[...] + recv_ref[k]

    return pl.pallas_call(
        body,
        out_shape=jax.ShapeDtypeStruct((m, n), x.dtype),
        in_specs=[pl.BlockSpec(memory_space=pltpu.VMEM)],
        out_specs=pl.BlockSpec(memory_space=pltpu.VMEM),
        scratch_shapes=[
            pltpu.VMEM((LOG2_N, m, n), x.dtype),
            pltpu.SemaphoreType.DMA((LOG2_N,)),
            pltpu.SemaphoreType.DMA((LOG2_N,)),
        ],
        compiler_params=pltpu.CompilerParams(collective_id=0),
    )(x)


# device time: 22031 ns/iter; 1.2699x vs baseline; 1.2699x over previous
import jax
import jax.numpy as jnp
from jax import lax
from jax.experimental import pallas as pl
from jax.experimental.pallas import tpu as pltpu

N_DEV = 16
LOG2_N = 4
N_CHUNK = 4


def kernel(x):
    m, n = x.shape
    rows = m // N_CHUNK

    def body(x_ref, out_ref, recv_ref, send_sems, recv_sems):
        my = lax.axis_index("i")

        barrier = pltpu.get_barrier_semaphore()
        for k in range(LOG2_N):
            partner = my ^ (1 << k)
            pl.semaphore_signal(
                barrier,
                inc=1,
                device_id=(partner,),
                device_id_type=pl.DeviceIdType.MESH,
            )
        pl.semaphore_wait(barrier, LOG2_N)

        out_ref[...] = x_ref[...]

        def make(k, c):
            partner = my ^ (1 << k)
            return pltpu.make_async_remote_copy(
                src_ref=out_ref.at[pl.ds(c * rows, rows), :],
                dst_ref=recv_ref.at[k, c],
                send_sem=send_sems.at[k, c],
                recv_sem=recv_sems.at[k, c],
                device_id=(partner,),
                device_id_type=pl.DeviceIdType.MESH,
            )

        descs = [[make(k, c) for c in range(N_CHUNK)] for k in range(LOG2_N)]
        for c in range(N_CHUNK):
            descs[0][c].start()
        for k in range(1, LOG2_N):
            for c in range(N_CHUNK):
                descs[k - 1][c].wait()
                out_ref[pl.ds(c * rows, rows), :] = (
                    out_ref[pl.ds(c * rows, rows), :] + recv_ref[k - 1, c]
                )
                descs[k][c].start()
        for c in range(N_CHUNK):
            descs[LOG2_N - 1][c].wait()
            out_ref[pl.ds(c * rows, rows), :] = (
                out_ref[pl.ds(c * rows, rows), :] + recv_ref[LOG2_N - 1, c]
            )

    return pl.pallas_call(
        body,
        out_shape=jax.ShapeDtypeStruct((m, n), x.dtype),
        in_specs=[pl.BlockSpec(memory_space=pltpu.VMEM)],
        out_specs=pl.BlockSpec(memory_space=pltpu.VMEM),
        scratch_shapes=[
            pltpu.VMEM((LOG2_N, N_CHUNK, rows, n), x.dtype),
            pltpu.SemaphoreType.DMA((LOG2_N, N_CHUNK)),
            pltpu.SemaphoreType.DMA((LOG2_N, N_CHUNK)),
        ],
        compiler_params=pltpu.CompilerParams(collective_id=0),
    )(x)


# device time: 19581 ns/iter; 1.4288x vs baseline; 1.1251x over previous
import jax
import jax.numpy as jnp
from jax import lax
from jax.experimental import pallas as pl
from jax.experimental.pallas import tpu as pltpu

N_DEV = 16


def kernel(x):
    m, n = x.shape
    rows = m // N_DEV

    def body(x_ref, out_ref, rs_buf, ag_buf, red_ref,
             rs_send, rs_recv, ag_send, ag_recv):
        my = lax.axis_index("i")

        barrier = pltpu.get_barrier_semaphore()
        for d in range(1, N_DEV):
            pl.semaphore_signal(
                barrier,
                inc=1,
                device_id=(my ^ d,),
                device_id_type=pl.DeviceIdType.MESH,
            )
        pl.semaphore_wait(barrier, N_DEV - 1)

        rs = []
        for d in range(1, N_DEV):
            peer = my ^ d
            r = pltpu.make_async_remote_copy(
                src_ref=x_ref.at[pl.ds(peer * rows, rows), :],
                dst_ref=rs_buf.at[d],
                send_sem=rs_send.at[d],
                recv_sem=rs_recv.at[d],
                device_id=(peer,),
                device_id_type=pl.DeviceIdType.MESH,
            )
            r.start()
            rs.append(r)

        red_ref[...] = x_ref[pl.ds(my * rows, rows), :]
        for d in range(1, N_DEV):
            rs[d - 1].wait_recv()
            red_ref[...] = red_ref[...] + rs_buf[d]

        ag = []
        for d in range(1, N_DEV):
            peer = my ^ d
            r = pltpu.make_async_remote_copy(
                src_ref=red_ref,
                dst_ref=ag_buf.at[d],
                send_sem=ag_send.at[d],
                recv_sem=ag_recv.at[d],
                device_id=(peer,),
                device_id_type=pl.DeviceIdType.MESH,
            )
            r.start()
            ag.append(r)

        out_ref[pl.ds(my * rows, rows), :] = red_ref[...]
        for d in range(1, N_DEV):
            ag[d - 1].wait_recv()
            out_ref[pl.ds((my ^ d) * rows, rows), :] = ag_buf[d]

        for d in range(1, N_DEV):
            rs[d - 1].wait_send()
            ag[d - 1].wait_send()

    return pl.pallas_call(
        body,
        out_shape=jax.ShapeDtypeStruct((m, n), x.dtype),
        in_specs=[pl.BlockSpec(memory_space=pltpu.VMEM)],
        out_specs=pl.BlockSpec(memory_space=pltpu.VMEM),
        scratch_shapes=[
            pltpu.VMEM((N_DEV, rows, n), x.dtype),
            pltpu.VMEM((N_DEV, rows, n), x.dtype),
            pltpu.VMEM((rows, n), x.dtype),
            pltpu.SemaphoreType.DMA((N_DEV,)),
            pltpu.SemaphoreType.DMA((N_DEV,)),
            pltpu.SemaphoreType.DMA((N_DEV,)),
            pltpu.SemaphoreType.DMA((N_DEV,)),
        ],
        compiler_params=pltpu.CompilerParams(collective_id=0),
    )(x)


# device time: 4906 ns/iter; 5.7028x vs baseline; 3.9912x over previous
import os

import jax
import jax.numpy as jnp
from jax import lax
from jax.experimental import pallas as pl
from jax.experimental.pallas import tpu as pltpu

N_DEV = 16

_PHASES = os.environ.get("ABLATE_PHASES", "")
_DO_RS = _PHASES in ("", "rs")
_DO_AG = _PHASES == ""


def kernel(x):
    m, n = x.shape
    rows = m // N_DEV

    def body(x_ref, out_ref, rs_buf, red_ref,
             rs_send, rs_recv, ag_send, ag_recv):
        my = lax.axis_index("i")

        barrier = pltpu.get_barrier_semaphore()
        if os.environ.get("ABLATE_BARRIER") == "neighbor15":
            for _ in range(N_DEV - 1):
                pl.semaphore_signal(
                    barrier,
                    inc=1,
                    device_id=(my ^ 1,),
                    device_id_type=pl.DeviceIdType.MESH,
                )
        else:
            for d in range(1, N_DEV):
                pl.semaphore_signal(
                    barrier,
                    inc=1,
                    device_id=(my ^ d,),
                    device_id_type=pl.DeviceIdType.MESH,
                )
        pl.semaphore_wait(barrier, N_DEV - 1)

        if not _DO_RS:
            out_ref[...] = x_ref[...]
            return

        rs = [None] * N_DEV
        for d in range(N_DEV - 1, 0, -1):
            peer = my ^ d
            r = pltpu.make_async_remote_copy(
                src_ref=x_ref.at[pl.ds(peer * rows, rows), :],
                dst_ref=rs_buf.at[d],
                send_sem=rs_send.at[d],
                recv_sem=rs_recv.at[d],
                device_id=(peer,),
                device_id_type=pl.DeviceIdType.MESH,
            )
            r.start()
            rs[d] = r

        rs_buf[0] = x_ref[pl.ds(my * rows, rows), :]
        for d in range(1, N_DEV):
            rs[d].wait_recv()
        rs_buf[0:8] = rs_buf[0:8] + rs_buf[8:16]
        rs_buf[0:4] = rs_buf[0:4] + rs_buf[4:8]
        rs_buf[0:2] = rs_buf[0:2] + rs_buf[2:4]
        red_ref[...] = rs_buf[0] + rs_buf[1]

        if not _DO_AG:
            out_ref[...] = x_ref[...]
            out_ref[pl.ds(my * rows, rows), :] = red_ref[...]
            for d in range(1, N_DEV):
                rs[d].wait_send()
            return

        ag = []
        for d in range(N_DEV - 1, 0, -1):
            peer = my ^ d
            r = pltpu.make_async_remote_copy(
                src_ref=red_ref,
                dst_ref=out_ref.at[pl.ds(my * rows, rows), :],
                send_sem=ag_send.at[d],
                recv_sem=ag_recv.at[d],
                device_id=(peer,),
                device_id_type=pl.DeviceIdType.MESH,
            )
            r.start()
            ag.append(r)

        out_ref[pl.ds(my * rows, rows), :] = red_ref[...]
        for r in ag:
            r.wait_recv()

        for d in range(1, N_DEV):
            rs[d].wait_send()
        for r in ag:
            r.wait_send()

    return pl.pallas_call(
        body,
        out_shape=jax.ShapeDtypeStruct((m, n), x.dtype),
        in_specs=[pl.BlockSpec(memory_space=pltpu.VMEM)],
        out_specs=pl.BlockSpec(memory_space=pltpu.VMEM),
        scratch_shapes=[
            pltpu.VMEM((N_DEV, rows, n), x.dtype),
            pltpu.VMEM((rows, n), x.dtype),
            pltpu.SemaphoreType.DMA((N_DEV,)),
            pltpu.SemaphoreType.DMA((N_DEV,)),
            pltpu.SemaphoreType.DMA((N_DEV,)),
            pltpu.SemaphoreType.DMA((N_DEV,)),
        ],
        compiler_params=pltpu.CompilerParams(collective_id=0),
    )(x)
